# Initial kernel scaffold; baseline (speedup 1.0000x reference)
import jax
import jax.numpy as jnp
from jax.experimental import pallas as pl


def kernel(x, edge_index, W1, b1, W2, b2, W3, b3, Wc, bc):
    raise NotImplementedError("write your pallas kernel here")


# speedup vs baseline: 20.1736x; 20.1736x over previous
import functools

import jax
import jax.numpy as jnp
from jax import lax
from jax.experimental import pallas as pl
from jax.experimental.pallas import tpu as pltpu
from jax.experimental.pallas import tpu_sc as plsc

NC = 2
NS = 16
LANES = 128
SB = 16
SBE = 8
D = 16


def _mesh():
    return plsc.VectorSubcoreMesh(core_axis_name="c", subcore_axis_name="s")


_SC_PARAMS = pltpu.CompilerParams(use_tc_tiling_on_sc=False)


def _sc_degree(dst2d, n_pad):
    rows_p = dst2d.shape[0]
    rows_w = rows_p // (NC * NS)
    outer = rows_w // SB
    rps = n_pad // NS

    @functools.partial(
        pl.kernel,
        out_type=jax.ShapeDtypeStruct((NC * n_pad,), jnp.float32),
        mesh=_mesh(),
        compiler_params=_SC_PARAMS,
        scratch_types=[
            pltpu.VMEM((SB, LANES), jnp.int32),
            pltpu.VMEM((LANES,), jnp.float32),
            pltpu.VMEM((rps,), jnp.float32),
            pltpu.VMEM_SHARED((n_pad,), jnp.float32),
            pltpu.SemaphoreType.DMA,
        ],
    )
    def k(dst_hbm, out_hbm, dstv, ones, zb, acc, sem):
        c = lax.axis_index("c")
        s = lax.axis_index("s")

        @pl.loop(0, rps // 16)
        def _(i):
            zb[pl.ds(i * 16, 16)] = jnp.zeros((16,), jnp.float32)

        @pl.loop(0, LANES // 16)
        def _(i):
            ones[pl.ds(i * 16, 16)] = jnp.ones((16,), jnp.float32)

        pltpu.sync_copy(zb, acc.at[pl.ds(s * rps, rps)])
        plsc.subcore_barrier()

        base = (c * NS + s) * rows_w

        @pl.loop(0, outer)
        def _(t):
            pltpu.sync_copy(dst_hbm.at[pl.ds(base + t * SB, SB), :], dstv)
            adds = [
                pltpu.async_copy(ones, acc.at[dstv.at[j]], sem, add=True)
                for j in range(SB)
            ]
            for a in adds:
                a.wait()

        plsc.subcore_barrier()
        pltpu.sync_copy(acc.at[pl.ds(s * rps, rps)], zb)
        pltpu.sync_copy(zb, out_hbm.at[pl.ds(c * n_pad + s * rps, rps)])

    return k(dst2d).reshape(NC, n_pad)


def _sc_edge_pass(ht, src2d, dst2d):
    n_pad = ht.shape[0]
    rows_p = src2d.shape[0]
    rows_w = rows_p // (NC * NS)
    outer = rows_w // SBE
    rps = n_pad // NS
    zrows = rps // 17

    @functools.partial(
        pl.kernel,
        out_type=jax.ShapeDtypeStruct((NC, n_pad, D), jnp.float32),
        mesh=_mesh(),
        compiler_params=_SC_PARAMS,
        scratch_types=[
            pltpu.VMEM((SBE, LANES), jnp.int32),
            pltpu.VMEM((SBE, LANES), jnp.int32),
            pltpu.VMEM((SBE * LANES, D), jnp.float32),
            pltpu.VMEM((zrows, D), jnp.float32),
            pltpu.VMEM_SHARED((n_pad, D), jnp.float32),
            pltpu.SemaphoreType.DMA,
            pltpu.SemaphoreType.DMA,
        ],
    )
    def k(h_hbm, src_hbm, dst_hbm, out_hbm, srcv, dstv, rowsv, zb, acc,
          semg, sems):
        c = lax.axis_index("c")
        s = lax.axis_index("s")

        @pl.loop(0, zrows)
        def _(i):
            zb[i, :] = jnp.zeros((D,), jnp.float32)

        @pl.loop(0, rps // zrows)
        def _(i):
            pltpu.sync_copy(zb, acc.at[pl.ds(s * rps + i * zrows, zrows), :])

        plsc.subcore_barrier()

        base = (c * NS + s) * rows_w

        @pl.loop(0, outer)
        def _(t):
            rb = base + t * SBE
            pltpu.sync_copy(src_hbm.at[pl.ds(rb, SBE), :], srcv)
            pltpu.sync_copy(dst_hbm.at[pl.ds(rb, SBE), :], dstv)
            gathers = [
                pltpu.async_copy(h_hbm.at[srcv.at[j]],
                                 rowsv.at[pl.ds(j * LANES, LANES), :], semg)
                for j in range(SBE)
            ]
            for g in gathers:
                g.wait()
            scats = [
                pltpu.async_copy(rowsv.at[pl.ds(j * LANES, LANES), :],
                                 acc.at[dstv.at[j]], sems, add=True)
                for j in range(SBE)
            ]
            for g in scats:
                g.wait()

        plsc.subcore_barrier()

        @pl.loop(0, rps // zrows)
        def _(i):
            pltpu.sync_copy(acc.at[pl.ds(s * rps + i * zrows, zrows), :], zb)
            pltpu.sync_copy(zb,
                            out_hbm.at[c, pl.ds(s * rps + i * zrows, zrows), :])

    return k(ht, src2d, dst2d)


def _tc_matmul(a, w, bias=None):
    n_pad, kdim = a.shape
    f = w.shape[1]
    rb = n_pad // 16

    def body(*refs):
        if bias is None:
            ar, wr, o = refs
            o[...] = jnp.dot(ar[...], wr[...],
                             preferred_element_type=jnp.float32)
        else:
            ar, wr, br, o = refs
            o[...] = jnp.dot(ar[...], wr[...],
                             preferred_element_type=jnp.float32) + br[...]

    in_specs = [
        pl.BlockSpec((rb, kdim), lambda i: (i, 0)),
        pl.BlockSpec((kdim, f), lambda i: (0, 0)),
    ]
    args = [a, w]
    if bias is not None:
        in_specs.append(pl.BlockSpec((1, f), lambda i: (0, 0)))
        args.append(bias)
    return pl.pallas_call(
        body,
        grid=(16,),
        in_specs=in_specs,
        out_specs=pl.BlockSpec((rb, f), lambda i: (i, 0)),
        out_shape=jax.ShapeDtypeStruct((n_pad, f), jnp.float32),
    )(*args)


def _tc_scale(deg_t, h1raw):
    n_pad = h1raw.shape[0]
    rb = n_pad // 16

    def body(dref, href, dout, hout):
        dsum = dref[:, 0:1] + dref[:, 1:2] + 1.0
        dcol = lax.rsqrt(dsum)
        de = jnp.broadcast_to(dcol, (rb, D))
        dout[...] = de
        hout[...] = href[...] * de

    return pl.pallas_call(
        body,
        grid=(16,),
        in_specs=[
            pl.BlockSpec((rb, 2), lambda i: (i, 0)),
            pl.BlockSpec((rb, D), lambda i: (i, 0)),
        ],
        out_specs=[
            pl.BlockSpec((rb, D), lambda i: (i, 0)),
            pl.BlockSpec((rb, D), lambda i: (i, 0)),
        ],
        out_shape=[
            jax.ShapeDtypeStruct((n_pad, D), jnp.float32),
            jax.ShapeDtypeStruct((n_pad, D), jnp.float32),
        ],
    )(deg_t, h1raw)


def _tc_combine(p0p, p1p, htp, dp, be):
    m = p0p.shape[0]
    rb = m // 4

    def body(a, b, h, dd, br, tout, tdout):
        z = (a[...] + b[...] + h[...]) * dd[...] + br[...]
        t = jnp.tanh(z)
        tout[...] = t
        tdout[...] = t * dd[...]

    blk = pl.BlockSpec((rb, 128), lambda i: (i, 0))
    return pl.pallas_call(
        body,
        grid=(4,),
        in_specs=[blk, blk, blk, blk, pl.BlockSpec((1, 128), lambda i: (0, 0))],
        out_specs=[blk, blk],
        out_shape=[
            jax.ShapeDtypeStruct((m, 128), jnp.float32),
            jax.ShapeDtypeStruct((m, 128), jnp.float32),
        ],
    )(p0p, p1p, htp, dp, be)


def _pad_w(w):
    k, f = w.shape
    return jnp.zeros((k, D), jnp.float32).at[:, :f].set(w)


def _pad_b(b):
    bp = jnp.zeros((D,), jnp.float32).at[: b.shape[0]].set(b)
    return jnp.tile(bp, 128 // D).reshape(1, 128)


def kernel(x, edge_index, W1, b1, W2, b2, W3, b3, Wc, bc):
    n = x.shape[0]
    e = edge_index.shape[1]
    n_pad = -(-(n + 1) // 128) * 128
    m = n_pad * D // 128

    src = edge_index[0].astype(jnp.int32)
    dst = edge_index[1].astype(jnp.int32)

    rows = -(-e // LANES)
    rows_w = -(-rows // (NC * NS))
    rows_w = -(-rows_w // SB) * SB
    rows_p = rows_w * NC * NS
    pad = rows_p * LANES - e
    scratch_rows = n_pad - n
    src_p = jnp.concatenate([src, jnp.zeros((pad,), jnp.int32)])
    dst_p = jnp.concatenate(
        [dst, n + (jnp.arange(pad, dtype=jnp.int32) % scratch_rows)])
    src2d = src_p.reshape(rows_p, LANES)
    dst2d = dst_p.reshape(rows_p, LANES)

    xp = jnp.pad(x, ((0, n_pad - n), (0, 0)))
    w1p, w2p, w3p = _pad_w(W1), _pad_w(W2), _pad_w(W3)
    wcp = _pad_w(Wc)
    be1, be2, be3 = _pad_b(b1), _pad_b(b2), _pad_b(b3)

    h1raw = _tc_matmul(xp, w1p)
    degp = _sc_degree(dst2d, n_pad)

    deg_t = jnp.transpose(degp)
    dinv_e, h1t = _tc_scale(deg_t, h1raw)
    dp = dinv_e.reshape(m, 128)

    ht = h1t
    t_packed = None
    for wnext, be in ((w2p, be1), (w3p, be2), (None, be3)):
        p = _sc_edge_pass(ht, src2d, dst2d)
        t_packed, td_packed = _tc_combine(
            p[0].reshape(m, 128), p[1].reshape(m, 128),
            ht.reshape(m, 128), dp, be)
        if wnext is not None:
            ht = _tc_matmul(td_packed.reshape(n_pad, D), wnext)

    t3 = t_packed.reshape(n_pad, D)
    bcp = jnp.zeros((1, D), jnp.float32).at[0, : bc.shape[0]].set(bc)
    outp = _tc_matmul(t3, wcp, bcp)

    return (outp[:n, : Wc.shape[1]], t3[:n, : W3.shape[1]])


# speedup vs baseline: 20.1938x; 1.0010x over previous
import functools

import jax
import jax.numpy as jnp
from jax import lax
from jax.experimental import pallas as pl
from jax.experimental.pallas import tpu as pltpu
from jax.experimental.pallas import tpu_sc as plsc

NC = 2
NS = 16
LANES = 128
SB = 16
SBE = 8
D = 16


def _mesh():
    return plsc.VectorSubcoreMesh(core_axis_name="c", subcore_axis_name="s")


_SC_PARAMS = pltpu.CompilerParams(use_tc_tiling_on_sc=False)


def _sc_degree(dst_flat, n_pad):
    ep = dst_flat.shape[0]
    ew = ep // (NC * NS)
    ch = SB * LANES
    outer = ew // ch
    rps = n_pad // NS

    @functools.partial(
        pl.kernel,
        out_type=jax.ShapeDtypeStruct((NC * n_pad,), jnp.float32),
        mesh=_mesh(),
        compiler_params=_SC_PARAMS,
        scratch_types=[
            pltpu.VMEM((ch,), jnp.int32),
            pltpu.VMEM((ch,), jnp.float32),
            pltpu.VMEM((rps,), jnp.float32),
            pltpu.VMEM_SHARED((n_pad,), jnp.float32),
            pltpu.SemaphoreType.DMA,
        ],
    )
    def k(dst_hbm, out_hbm, dstv, ones, zb, acc, sem):
        c = lax.axis_index("c")
        s = lax.axis_index("s")

        @pl.loop(0, rps // 16)
        def _(i):
            zb[pl.ds(i * 16, 16)] = jnp.zeros((16,), jnp.float32)

        @pl.loop(0, ch // 16)
        def _(i):
            ones[pl.ds(i * 16, 16)] = jnp.ones((16,), jnp.float32)

        pltpu.sync_copy(zb, acc.at[pl.ds(s * rps, rps)])
        plsc.subcore_barrier()

        base = (c * NS + s) * ew

        @pl.loop(0, outer)
        def _(t):
            pltpu.sync_copy(dst_hbm.at[pl.ds(base + t * ch, ch)], dstv)
            pltpu.async_copy(ones, acc.at[dstv], sem, add=True).wait()

        plsc.subcore_barrier()
        pltpu.sync_copy(acc.at[pl.ds(s * rps, rps)], zb)
        pltpu.sync_copy(zb, out_hbm.at[pl.ds(c * n_pad + s * rps, rps)])

    return k(dst_flat).reshape(NC, n_pad)


def _sc_edge_pass(ht, src_flat, dst_flat):
    n_pad = ht.shape[0]
    ep = src_flat.shape[0]
    ew = ep // (NC * NS)
    ch = SBE * LANES
    outer = ew // ch
    rps = n_pad // NS
    zrows = rps // 17

    @functools.partial(
        pl.kernel,
        out_type=jax.ShapeDtypeStruct((NC, n_pad, D), jnp.float32),
        mesh=_mesh(),
        compiler_params=_SC_PARAMS,
        scratch_types=[
            pltpu.VMEM((ch,), jnp.int32),
            pltpu.VMEM((ch,), jnp.int32),
            pltpu.VMEM((ch, D), jnp.float32),
            pltpu.VMEM((zrows, D), jnp.float32),
            pltpu.VMEM_SHARED((n_pad, D), jnp.float32),
            pltpu.SemaphoreType.DMA,
            pltpu.SemaphoreType.DMA,
        ],
    )
    def k(h_hbm, src_hbm, dst_hbm, out_hbm, srcv, dstv, rowsv, zb, acc,
          semg, sems):
        c = lax.axis_index("c")
        s = lax.axis_index("s")

        @pl.loop(0, zrows)
        def _(i):
            zb[i, :] = jnp.zeros((D,), jnp.float32)

        @pl.loop(0, rps // zrows)
        def _(i):
            pltpu.sync_copy(zb, acc.at[pl.ds(s * rps + i * zrows, zrows), :])

        plsc.subcore_barrier()

        base = (c * NS + s) * ew

        @pl.loop(0, outer)
        def _(t):
            eb = base + t * ch
            pltpu.sync_copy(src_hbm.at[pl.ds(eb, ch)], srcv)
            pltpu.sync_copy(dst_hbm.at[pl.ds(eb, ch)], dstv)
            pltpu.async_copy(h_hbm.at[srcv], rowsv, semg).wait()
            pltpu.async_copy(rowsv, acc.at[dstv], sems, add=True).wait()

        plsc.subcore_barrier()

        @pl.loop(0, rps // zrows)
        def _(i):
            pltpu.sync_copy(acc.at[pl.ds(s * rps + i * zrows, zrows), :], zb)
            pltpu.sync_copy(zb,
                            out_hbm.at[c, pl.ds(s * rps + i * zrows, zrows), :])

    return k(ht, src_flat, dst_flat)


def _tc_matmul(a, w, bias=None):
    n_pad, kdim = a.shape
    f = w.shape[1]
    rb = n_pad // 16

    def body(*refs):
        if bias is None:
            ar, wr, o = refs
            o[...] = jnp.dot(ar[...], wr[...],
                             preferred_element_type=jnp.float32)
        else:
            ar, wr, br, o = refs
            o[...] = jnp.dot(ar[...], wr[...],
                             preferred_element_type=jnp.float32) + br[...]

    in_specs = [
        pl.BlockSpec((rb, kdim), lambda i: (i, 0)),
        pl.BlockSpec((kdim, f), lambda i: (0, 0)),
    ]
    args = [a, w]
    if bias is not None:
        in_specs.append(pl.BlockSpec((1, f), lambda i: (0, 0)))
        args.append(bias)
    return pl.pallas_call(
        body,
        grid=(16,),
        in_specs=in_specs,
        out_specs=pl.BlockSpec((rb, f), lambda i: (i, 0)),
        out_shape=jax.ShapeDtypeStruct((n_pad, f), jnp.float32),
    )(*args)


def _tc_scale(deg_t, h1raw):
    n_pad = h1raw.shape[0]
    rb = n_pad // 16

    def body(dref, href, dout, hout):
        dsum = dref[:, 0:1] + dref[:, 1:2] + 1.0
        dcol = lax.rsqrt(dsum)
        de = jnp.broadcast_to(dcol, (rb, D))
        dout[...] = de
        hout[...] = href[...] * de

    return pl.pallas_call(
        body,
        grid=(16,),
        in_specs=[
            pl.BlockSpec((rb, 2), lambda i: (i, 0)),
            pl.BlockSpec((rb, D), lambda i: (i, 0)),
        ],
        out_specs=[
            pl.BlockSpec((rb, D), lambda i: (i, 0)),
            pl.BlockSpec((rb, D), lambda i: (i, 0)),
        ],
        out_shape=[
            jax.ShapeDtypeStruct((n_pad, D), jnp.float32),
            jax.ShapeDtypeStruct((n_pad, D), jnp.float32),
        ],
    )(deg_t, h1raw)


def _tc_combine(p0p, p1p, htp, dp, be):
    m = p0p.shape[0]
    rb = m // 4

    def body(a, b, h, dd, br, tout, tdout):
        z = (a[...] + b[...] + h[...]) * dd[...] + br[...]
        t = jnp.tanh(z)
        tout[...] = t
        tdout[...] = t * dd[...]

    blk = pl.BlockSpec((rb, 128), lambda i: (i, 0))
    return pl.pallas_call(
        body,
        grid=(4,),
        in_specs=[blk, blk, blk, blk, pl.BlockSpec((1, 128), lambda i: (0, 0))],
        out_specs=[blk, blk],
        out_shape=[
            jax.ShapeDtypeStruct((m, 128), jnp.float32),
            jax.ShapeDtypeStruct((m, 128), jnp.float32),
        ],
    )(p0p, p1p, htp, dp, be)


def _pad_w(w):
    k, f = w.shape
    return jnp.zeros((k, D), jnp.float32).at[:, :f].set(w)


def _pad_b(b):
    bp = jnp.zeros((D,), jnp.float32).at[: b.shape[0]].set(b)
    return jnp.tile(bp, 128 // D).reshape(1, 128)


def kernel(x, edge_index, W1, b1, W2, b2, W3, b3, Wc, bc):
    n = x.shape[0]
    e = edge_index.shape[1]
    n_pad = -(-(n + 1) // 128) * 128
    m = n_pad * D // 128

    src = edge_index[0].astype(jnp.int32)
    dst = edge_index[1].astype(jnp.int32)

    rows = -(-e // LANES)
    rows_w = -(-rows // (NC * NS))
    rows_w = -(-rows_w // SB) * SB
    rows_p = rows_w * NC * NS
    pad = rows_p * LANES - e
    scratch_rows = n_pad - n
    src_p = jnp.concatenate([src, jnp.zeros((pad,), jnp.int32)])
    dst_p = jnp.concatenate(
        [dst, n + (jnp.arange(pad, dtype=jnp.int32) % scratch_rows)])

    xp = jnp.pad(x, ((0, n_pad - n), (0, 0)))
    w1p, w2p, w3p = _pad_w(W1), _pad_w(W2), _pad_w(W3)
    wcp = _pad_w(Wc)
    be1, be2, be3 = _pad_b(b1), _pad_b(b2), _pad_b(b3)

    h1raw = _tc_matmul(xp, w1p)
    degp = _sc_degree(dst_p, n_pad)

    deg_t = jnp.transpose(degp)
    dinv_e, h1t = _tc_scale(deg_t, h1raw)
    dp = dinv_e.reshape(m, 128)

    ht = h1t
    t_packed = None
    for wnext, be in ((w2p, be1), (w3p, be2), (None, be3)):
        p = _sc_edge_pass(ht, src_p, dst_p)
        t_packed, td_packed = _tc_combine(
            p[0].reshape(m, 128), p[1].reshape(m, 128),
            ht.reshape(m, 128), dp, be)
        if wnext is not None:
            ht = _tc_matmul(td_packed.reshape(n_pad, D), wnext)

    t3 = t_packed.reshape(n_pad, D)
    bcp = jnp.zeros((1, D), jnp.float32).at[0, : bc.shape[0]].set(bc)
    outp = _tc_matmul(t3, wcp, bcp)

    return (outp[:n, : Wc.shape[1]], t3[:n, : W3.shape[1]])
